# baseline (speedup 1.0000x reference)
import functools

import jax
import jax.numpy as jnp
from jax import lax
from jax.experimental import pallas as pl
from jax.experimental.pallas import tpu as pltpu
from jax.experimental.pallas import tpu_sc as plsc

N = 10000
E = 320000
D = 128
DH = D // 2
H = 128
T = 2
G = 8

NC = 2
NS = 16
NW = NC * NS
EPW = E // NW
KP = 64
EPWP = 10240
NCHUNKP = EPWP // KP
NPAIR = NCHUNKP // 2
NPAD = 10112
RPT = NPAD // NS


def _stage1_body(x_ref, w1_ref, b1_ref, g1_ref, be1_ref, wl0_ref, bl0_ref,
                 flo_ref, fhi_ref, np0_ref):
    h = lax.dot_general(x_ref[...], w1_ref[...], (((1,), (1,)), ((), ())),
                        precision=lax.Precision.HIGHEST)
    h = h + b1_ref[...]
    mu = jnp.mean(h, axis=0, keepdims=True)
    var = jnp.mean((h - mu) ** 2, axis=0, keepdims=True)
    hn = (h - mu) * lax.rsqrt(var + 1e-5)
    f = jnp.maximum(g1_ref[...] * hn + be1_ref[...], 0.0)
    np0_ref[...] = lax.dot_general(
        f, wl0_ref[...], (((1,), (1,)), ((), ())),
        precision=lax.Precision.HIGHEST) + bl0_ref[...]
    flo_ref[0:N, :] = f[:, 0:DH]
    fhi_ref[0:N, :] = f[:, DH:D]
    flo_ref[N:NPAD, :] = jnp.zeros((NPAD - N, DH), jnp.float32)
    fhi_ref[N:NPAD, :] = jnp.zeros((NPAD - N, DH), jnp.float32)


_stage1 = pl.pallas_call(
    _stage1_body,
    out_shape=[
        jax.ShapeDtypeStruct((NPAD, DH), jnp.float32),
        jax.ShapeDtypeStruct((NPAD, DH), jnp.float32),
        jax.ShapeDtypeStruct((N, T), jnp.float32),
    ],
)


def _sc_agg_body(src_hbm, dst_hbm, flo_hbm, fhi_hbm, zeros_hbm, out_hbm,
                 sidx, didx, bufs, feat_sh, accum,
                 isem_a, isem_b, gsem_a, gsem_b, ssem_a, ssem_b):
    c = lax.axis_index("c")
    s = lax.axis_index("s")
    wid = c * NS + s
    isem = (isem_a, isem_b)
    gsem = (gsem_a, gsem_b)
    ssem = (ssem_a, ssem_b)

    def issue_sidx(ch, p):
        pltpu.async_copy(src_hbm.at[wid, ch], sidx.at[p], isem[p])

    def wait_sidx(p):
        pltpu.make_async_copy(src_hbm.at[wid, 0], sidx.at[p], isem[p]).wait()

    def issue_gather(p, sem):
        pltpu.async_copy(feat_sh.at[sidx.at[p]], bufs.at[p], sem)

    def issue_scatter(ch, p, sem):
        pltpu.async_copy(bufs.at[p], accum.at[didx.at[ch]], sem, add=True)

    def wait_rows(p, sem):
        pltpu.make_async_copy(feat_sh.at[sidx.at[0]], bufs.at[p], sem).wait()

    def step(ch, p):
        q = 1 - p

        @pl.when(ch + 1 < NCHUNKP)
        def _():
            wait_sidx(q)

        @pl.when(ch >= 2)
        def _():
            wait_rows(q, ssem[q])

        @pl.when(ch + 1 < NCHUNKP)
        def _():
            issue_gather(q, gsem[q])
        wait_rows(p, gsem[p])

        @pl.when(ch + 2 < NCHUNKP)
        def _():
            issue_sidx(ch + 2, p)
        issue_scatter(ch, p, ssem[p])

    def run_pass(k, fsrc_hbm):
        z = pltpu.async_copy(zeros_hbm.at[pl.ds(s * RPT, RPT)],
                             accum.at[pl.ds(s * RPT, RPT)], isem_a)
        f = pltpu.async_copy(fsrc_hbm.at[pl.ds(s * RPT, RPT)],
                             feat_sh.at[pl.ds(s * RPT, RPT)], isem_b)
        s0 = pltpu.async_copy(src_hbm.at[wid, 0], sidx.at[0], gsem_a)
        if k == 0:
            pltpu.async_copy(dst_hbm.at[wid], didx, gsem_b).wait()
        z.wait()
        f.wait()
        s0.wait()
        plsc.subcore_barrier()

        issue_gather(0, gsem[0])
        issue_sidx(1, 1)

        def pair(t, carry):
            step(2 * t, 0)
            step(2 * t + 1, 1)
            return carry

        lax.fori_loop(0, NPAIR, pair, 0)
        wait_rows(0, ssem[0])
        wait_rows(1, ssem[1])
        plsc.subcore_barrier()

        pltpu.sync_copy(
            accum.at[pl.ds(s * RPT, RPT)],
            out_hbm.at[pl.ds((2 * c + k) * NPAD + s * RPT, RPT)])

    run_pass(0, flo_hbm)
    run_pass(1, fhi_hbm)


_sc_agg = functools.partial(
    pl.kernel,
    out_type=jax.ShapeDtypeStruct((4 * NPAD, DH), jnp.float32),
    mesh=plsc.VectorSubcoreMesh(core_axis_name="c", subcore_axis_name="s",
                                num_cores=NC, num_subcores=NS),
    compiler_params=pltpu.CompilerParams(use_tc_tiling_on_sc=False),
    scratch_types=[
        pltpu.VMEM((2, KP), jnp.int32),
        pltpu.VMEM((NCHUNKP, KP), jnp.int32),
        pltpu.VMEM((2, KP, DH), jnp.float32),
        pltpu.VMEM_SHARED((NPAD, DH), jnp.float32),
        pltpu.VMEM_SHARED((NPAD, DH), jnp.float32),
        pltpu.SemaphoreType.DMA,
        pltpu.SemaphoreType.DMA,
        pltpu.SemaphoreType.DMA,
        pltpu.SemaphoreType.DMA,
        pltpu.SemaphoreType.DMA,
        pltpu.SemaphoreType.DMA,
    ],
)(_sc_agg_body)


def _stage2a_body(flo_ref, fhi_ref, agg_ref, wc_ref, bc_ref, gc_ref, bec_ref,
                  wl1_ref, bl1_ref, np1_ref):
    h_lo = (flo_ref[0:N, :] + agg_ref[0:N, :]
            + agg_ref[2 * NPAD:2 * NPAD + N, :])
    h_hi = (fhi_ref[0:N, :] + agg_ref[NPAD:NPAD + N, :]
            + agg_ref[3 * NPAD:3 * NPAD + N, :])
    h = (lax.dot_general(h_lo, wc_ref[:, 0:DH], (((1,), (1,)), ((), ())),
                         precision=lax.Precision.HIGHEST)
         + lax.dot_general(h_hi, wc_ref[:, DH:D], (((1,), (1,)), ((), ())),
                           precision=lax.Precision.HIGHEST))
    h = h + bc_ref[...]
    mu = jnp.mean(h, axis=0, keepdims=True)
    var = jnp.mean((h - mu) ** 2, axis=0, keepdims=True)
    hn = (h - mu) * lax.rsqrt(var + 1e-5)
    f2 = jnp.maximum(gc_ref[...] * hn + bec_ref[...], 0.0)
    np1_ref[...] = lax.dot_general(
        f2, wl1_ref[...], (((1,), (1,)), ((), ())),
        precision=lax.Precision.HIGHEST) + bl1_ref[...]


_stage2a = pl.pallas_call(
    _stage2a_body,
    out_shape=jax.ShapeDtypeStruct((N, T), jnp.float32),
)


def _stage2b_body(np0_ref, np1_ref, batch_ref, np_ref, wsi_ref):
    np0 = np0_ref[...]
    np1 = np1_ref[...]
    np_ref[...] = np0 + np1

    mask = batch_ref[...] == lax.broadcasted_iota(jnp.int32, (1, G), 1)
    maskneg = jnp.where(mask, 0.0, -jnp.inf)
    rows = []
    for t in range(T):
        m0 = jnp.max(np0[:, t:t + 1] + maskneg, axis=0, keepdims=True)
        m1 = jnp.max(np1[:, t:t + 1] + maskneg, axis=0, keepdims=True)
        rows.append(m0 + m1)
    wsi_ref[...] = jnp.concatenate(rows, axis=0)


_stage2b = pl.pallas_call(
    _stage2b_body,
    out_shape=[
        jax.ShapeDtypeStruct((N, T), jnp.float32),
        jax.ShapeDtypeStruct((T, G), jnp.float32),
    ],
)


def kernel(x, W1, b1, g1, be1, Wl0, bl0, Wc, bc, gc, bec, Wl1, bl1,
           edge_index, batch):
    src = edge_index[0]
    dst = edge_index[1]
    pad = EPWP - EPW
    srcp = jnp.concatenate(
        [src.reshape(NW, EPW), jnp.zeros((NW, pad), jnp.int32)],
        axis=1).reshape(NW, NCHUNKP, KP)
    dstp = jnp.concatenate(
        [dst.reshape(NW, EPW), jnp.full((NW, pad), N, jnp.int32)],
        axis=1).reshape(NW, NCHUNKP, KP)
    flo, fhi, np0 = _stage1(x, W1, b1.reshape(1, H), g1.reshape(1, H),
                            be1.reshape(1, H), Wl0, bl0.reshape(1, T))
    zeros = jnp.zeros((NPAD, DH), jnp.float32)
    agg4 = _sc_agg(srcp, dstp, flo, fhi, zeros)
    np1 = _stage2a(flo, fhi, agg4, Wc, bc.reshape(1, H), gc.reshape(1, H),
                   bec.reshape(1, H), Wl1, bl1.reshape(1, T))
    node_pred, wsi_t = _stage2b(np0, np1, batch.reshape(N, 1))
    return (wsi_t.T, node_pred)
